# Initial kernel scaffold; baseline (speedup 1.0000x reference)
import jax
import jax.numpy as jnp
from jax.experimental import pallas as pl


def kernel(llr_ch):
    raise NotImplementedError("write your pallas kernel here")


# speedup vs baseline: 74.9753x; 74.9753x over previous
import functools

import jax
import jax.numpy as jnp
import numpy as np
from jax.experimental import pallas as pl

_N = 128
_N_STAGES = 7
_L = 8
_LLR_MAX = 30.0
_BATCH_TILE = 256

_FROZEN = np.array([1 if bin(i).count("1") <= 3 else 0 for i in range(_N)])
_K_INFO = int(_N - _FROZEN.sum())


def _cn_op(x, y):
    x = jnp.clip(x, -_LLR_MAX, _LLR_MAX)
    y = jnp.clip(y, -_LLR_MAX, _LLR_MAX)
    return jnp.logaddexp(0.0, x + y) - jnp.logaddexp(x, y)


def _vn_op(x, y, u):
    return (1.0 - 2.0 * u) * x + y


def _keepmin_masks(bt):
    pos = jax.lax.broadcasted_iota(jnp.int32, (16, bt), 0)
    masks = {}
    k = 2
    while k <= 16:
        j = k // 2
        while j >= 1:
            masks[(k, j)] = ((pos & j) == 0) == ((pos & k) == 0)
            j //= 2
        k *= 2
    return masks


def _bitonic_16(keys, km_masks):
    n = 16
    k = 2
    while k <= n:
        j = k // 2
        while j >= 1:
            x = keys.reshape(n // (2 * j), 2, j, keys.shape[-1])
            pk = jnp.concatenate([x[:, 1:2], x[:, 0:1]], axis=1).reshape(keys.shape)
            lo = jnp.minimum(keys, pk)
            hi = jnp.maximum(keys, pk)
            keys = jnp.where(km_masks[(k, j)], lo, hi)
            j //= 2
        k *= 2
    return keys


def _scl_tile(llr_ref, out_ref):
    bt = llr_ref.shape[1]
    f32 = jnp.float32

    llr_c = {_N_STAGES: llr_ref[...][None]}
    ulbuf = {}
    lane_l = jax.lax.broadcasted_iota(jnp.int32, (_L, bt), 0)
    pm = jnp.where(lane_l == 0, 0.0, 1.0e9).astype(f32)
    km_masks = _keepmin_masks(bt)
    cidx = jax.lax.broadcasted_iota(jnp.int32, (2 * _L, bt), 0)
    recs = []

    def gather_paths(arr, masks):
        if arr.shape[0] == 1:
            return arr
        acc = jnp.zeros_like(arr)
        for src in range(_L):
            acc = jnp.where(masks[src], arr[src][None], acc)
        return acc

    def leaf(i):
        nonlocal pm
        llr_i = jnp.clip(llr_c[0][:, 0, :], -_LLR_MAX, _LLR_MAX)
        del llr_c[0]
        if _FROZEN[i]:
            pm = pm + jax.nn.softplus(-llr_i)
            return jnp.zeros((1, 1, bt), f32)
        pm0 = pm + jax.nn.softplus(-llr_i)
        pm1 = pm + jax.nn.softplus(llr_i)
        cand = jnp.concatenate([pm0, pm1], axis=0)
        cbits = jax.lax.bitcast_convert_type(cand, jnp.int32)
        key = (cbits & -16) | cidx
        key8 = _bitonic_16(key, km_masks)[:_L]
        idx8 = key8 & 15
        pidx = idx8 & (_L - 1)
        u_new = (idx8 >= _L).astype(f32)
        pm_new = jnp.zeros((_L, bt), f32)
        for c in range(2 * _L):
            pm_new = jnp.where(idx8 == c, cand[c][None], pm_new)
        pm = pm_new
        masks = [(pidx == src)[:, None, :] for src in range(_L)]
        for s in list(llr_c):
            llr_c[s] = gather_paths(llr_c[s], masks)
        for s in list(ulbuf):
            ulbuf[s] = gather_paths(ulbuf[s], masks)
        recs.append((pidx, u_new))
        return u_new[:, None, :]

    def dec(start, n_sub, s):
        if n_sub == 1:
            return leaf(start)
        half = n_sub // 2
        x = llr_c[s][:, :half]
        y = llr_c[s][:, half:]
        llr_c[s - 1] = _cn_op(x, y)
        u_left = dec(start, half, s - 1)
        ulbuf[s - 1] = u_left
        x = llr_c[s][:, :half]
        y = llr_c[s][:, half:]
        llr_c[s - 1] = _vn_op(x, y, ulbuf[s - 1])
        del llr_c[s]
        u_right = dec(start + half, half, s - 1)
        u_left = ulbuf[s - 1]
        del ulbuf[s - 1]
        p = max(u_left.shape[0], u_right.shape[0])
        u_par = jnp.concatenate(
            [
                jnp.broadcast_to(jnp.mod(u_left + u_right, 2.0), (p, half, bt)),
                jnp.broadcast_to(u_right, (p, half, bt)),
            ],
            axis=1,
        )
        return u_par

    dec(0, _N, _N_STAGES)

    best = jnp.zeros((bt,), jnp.int32)
    bestv = pm[0]
    for l in range(1, _L):
        m = pm[l] < bestv
        bestv = jnp.where(m, pm[l], bestv)
        best = jnp.where(m, l, best)

    bits = [None] * len(recs)
    b = best
    for d in range(len(recs) - 1, -1, -1):
        pidx, u_new = recs[d]
        u_d = jnp.zeros((bt,), f32)
        b_prev = jnp.zeros((bt,), jnp.int32)
        for l in range(_L):
            m = b == l
            u_d = jnp.where(m, u_new[l], u_d)
            b_prev = jnp.where(m, pidx[l], b_prev)
        bits[d] = u_d
        b = b_prev
    out_ref[...] = jnp.stack(bits, axis=0)


@jax.jit
def _decode(llr_t):
    return pl.pallas_call(
        _scl_tile,
        grid=(llr_t.shape[1] // _BATCH_TILE,),
        in_specs=[pl.BlockSpec((_N, _BATCH_TILE), lambda ib: (0, ib))],
        out_specs=pl.BlockSpec((_K_INFO, _BATCH_TILE), lambda ib: (0, ib)),
        out_shape=jax.ShapeDtypeStruct((_K_INFO, llr_t.shape[1]), jnp.float32),
    )(llr_t)


def kernel(llr_ch):
    llr_t = llr_ch.T
    return _decode(llr_t).T


# speedup vs baseline: 82.6702x; 1.1026x over previous
import functools

import jax
import jax.numpy as jnp
import numpy as np
from jax.experimental import pallas as pl

_N = 128
_N_STAGES = 7
_L = 8
_LLR_MAX = 30.0
_BATCH_TILE = 512

_FROZEN = np.array([1 if bin(i).count("1") <= 3 else 0 for i in range(_N)])
_K_INFO = int(_N - _FROZEN.sum())


def _cn_op(x, y):
    x = jnp.clip(x, -_LLR_MAX, _LLR_MAX)
    y = jnp.clip(y, -_LLR_MAX, _LLR_MAX)
    return jnp.logaddexp(0.0, x + y) - jnp.logaddexp(x, y)


def _vn_op(x, y, u):
    return (1.0 - 2.0 * u) * x + y


def _keepmin_masks(bt):
    pos = jax.lax.broadcasted_iota(jnp.int32, (16, bt), 0)
    masks = {}
    k = 2
    while k <= 16:
        j = k // 2
        while j >= 1:
            masks[(k, j)] = ((pos & j) == 0) == ((pos & k) == 0)
            j //= 2
        k *= 2
    return masks


def _bitonic_16(keys, km_masks):
    n = 16
    k = 2
    while k <= n:
        j = k // 2
        while j >= 1:
            x = keys.reshape(n // (2 * j), 2, j, keys.shape[-1])
            pk = jnp.concatenate([x[:, 1:2], x[:, 0:1]], axis=1).reshape(keys.shape)
            lo = jnp.minimum(keys, pk)
            hi = jnp.maximum(keys, pk)
            keys = jnp.where(km_masks[(k, j)], lo, hi)
            j //= 2
        k *= 2
    return keys


def _scl_tile(llr_ref, out_ref):
    bt = llr_ref.shape[1]
    f32 = jnp.float32

    llr_c = {_N_STAGES: llr_ref[...][None]}
    ulbuf = {}
    lane_l = jax.lax.broadcasted_iota(jnp.int32, (_L, bt), 0)
    pm = jnp.where(lane_l == 0, 0.0, 1.0e9).astype(f32)
    km_masks = _keepmin_masks(bt)
    cidx = jax.lax.broadcasted_iota(jnp.int32, (2 * _L, bt), 0)
    recs = []

    def gather_paths(arr, masks):
        if arr.shape[0] == 1:
            return arr
        acc = jnp.zeros_like(arr)
        for src in range(_L):
            acc = jnp.where(masks[src], arr[src][None], acc)
        return acc

    def leaf(i):
        nonlocal pm
        llr_i = jnp.clip(llr_c[0][:, 0, :], -_LLR_MAX, _LLR_MAX)
        del llr_c[0]
        if _FROZEN[i]:
            pm = pm + jax.nn.softplus(-llr_i)
            return jnp.zeros((1, 1, bt), f32)
        pm0 = pm + jax.nn.softplus(-llr_i)
        pm1 = pm + jax.nn.softplus(llr_i)
        cand = jnp.concatenate([pm0, pm1], axis=0)
        cbits = jax.lax.bitcast_convert_type(cand, jnp.int32)
        key = (cbits & -16) | cidx
        key8 = _bitonic_16(key, km_masks)[:_L]
        idx8 = key8 & 15
        pidx = idx8 & (_L - 1)
        u_new = (idx8 >= _L).astype(f32)
        pm_new = jnp.zeros((_L, bt), f32)
        for c in range(2 * _L):
            pm_new = jnp.where(idx8 == c, cand[c][None], pm_new)
        pm = pm_new
        masks = [(pidx == src)[:, None, :] for src in range(_L)]
        for s in list(llr_c):
            llr_c[s] = gather_paths(llr_c[s], masks)
        for s in list(ulbuf):
            ulbuf[s] = gather_paths(ulbuf[s], masks)
        recs.append((pidx, u_new))
        return u_new[:, None, :]

    def dec(start, n_sub, s):
        if n_sub == 1:
            return leaf(start)
        half = n_sub // 2
        x = llr_c[s][:, :half]
        y = llr_c[s][:, half:]
        llr_c[s - 1] = _cn_op(x, y)
        u_left = dec(start, half, s - 1)
        ulbuf[s - 1] = u_left
        x = llr_c[s][:, :half]
        y = llr_c[s][:, half:]
        llr_c[s - 1] = _vn_op(x, y, ulbuf[s - 1])
        del llr_c[s]
        u_right = dec(start + half, half, s - 1)
        u_left = ulbuf[s - 1]
        del ulbuf[s - 1]
        p = max(u_left.shape[0], u_right.shape[0])
        u_par = jnp.concatenate(
            [
                jnp.broadcast_to(jnp.mod(u_left + u_right, 2.0), (p, half, bt)),
                jnp.broadcast_to(u_right, (p, half, bt)),
            ],
            axis=1,
        )
        return u_par

    dec(0, _N, _N_STAGES)

    best = jnp.zeros((bt,), jnp.int32)
    bestv = pm[0]
    for l in range(1, _L):
        m = pm[l] < bestv
        bestv = jnp.where(m, pm[l], bestv)
        best = jnp.where(m, l, best)

    bits = [None] * len(recs)
    b = best
    for d in range(len(recs) - 1, -1, -1):
        pidx, u_new = recs[d]
        u_d = jnp.zeros((bt,), f32)
        b_prev = jnp.zeros((bt,), jnp.int32)
        for l in range(_L):
            m = b == l
            u_d = jnp.where(m, u_new[l], u_d)
            b_prev = jnp.where(m, pidx[l], b_prev)
        bits[d] = u_d
        b = b_prev
    out_ref[...] = jnp.stack(bits, axis=0)


@jax.jit
def _decode(llr_t):
    return pl.pallas_call(
        _scl_tile,
        grid=(llr_t.shape[1] // _BATCH_TILE,),
        in_specs=[pl.BlockSpec((_N, _BATCH_TILE), lambda ib: (0, ib))],
        out_specs=pl.BlockSpec((_K_INFO, _BATCH_TILE), lambda ib: (0, ib)),
        out_shape=jax.ShapeDtypeStruct((_K_INFO, llr_t.shape[1]), jnp.float32),
    )(llr_t)


def kernel(llr_ch):
    llr_t = llr_ch.T
    return _decode(llr_t).T


# speedup vs baseline: 88.3440x; 1.0686x over previous
import functools

import jax
import jax.numpy as jnp
import numpy as np
from jax.experimental import pallas as pl

_N = 128
_N_STAGES = 7
_L = 8
_LLR_MAX = 30.0
_BATCH_TILE = 1024

_FROZEN = np.array([1 if bin(i).count("1") <= 3 else 0 for i in range(_N)])
_K_INFO = int(_N - _FROZEN.sum())


def _cn_op(x, y):
    x = jnp.clip(x, -_LLR_MAX, _LLR_MAX)
    y = jnp.clip(y, -_LLR_MAX, _LLR_MAX)
    return jnp.logaddexp(0.0, x + y) - jnp.logaddexp(x, y)


def _vn_op(x, y, u):
    return (1.0 - 2.0 * u) * x + y


def _keepmin_masks(bt):
    pos = jax.lax.broadcasted_iota(jnp.int32, (16, bt), 0)
    masks = {}
    k = 2
    while k <= 16:
        j = k // 2
        while j >= 1:
            masks[(k, j)] = ((pos & j) == 0) == ((pos & k) == 0)
            j //= 2
        k *= 2
    return masks


def _bitonic_16(keys, km_masks):
    n = 16
    k = 2
    while k <= n:
        j = k // 2
        while j >= 1:
            x = keys.reshape(n // (2 * j), 2, j, keys.shape[-1])
            pk = jnp.concatenate([x[:, 1:2], x[:, 0:1]], axis=1).reshape(keys.shape)
            lo = jnp.minimum(keys, pk)
            hi = jnp.maximum(keys, pk)
            keys = jnp.where(km_masks[(k, j)], lo, hi)
            j //= 2
        k *= 2
    return keys


def _scl_tile(llr_ref, out_ref):
    bt = llr_ref.shape[1]
    f32 = jnp.float32

    llr_c = {_N_STAGES: llr_ref[...][None]}
    ulbuf = {}
    lane_l = jax.lax.broadcasted_iota(jnp.int32, (_L, bt), 0)
    pm = jnp.where(lane_l == 0, 0.0, 1.0e9).astype(f32)
    km_masks = _keepmin_masks(bt)
    cidx = jax.lax.broadcasted_iota(jnp.int32, (2 * _L, bt), 0)
    recs = []

    def gather_paths(arr, masks):
        if arr.shape[0] == 1:
            return arr
        acc = jnp.zeros_like(arr)
        for src in range(_L):
            acc = jnp.where(masks[src], arr[src][None], acc)
        return acc

    def leaf(i):
        nonlocal pm
        llr_i = jnp.clip(llr_c[0][:, 0, :], -_LLR_MAX, _LLR_MAX)
        del llr_c[0]
        if _FROZEN[i]:
            pm = pm + jax.nn.softplus(-llr_i)
            return jnp.zeros((1, 1, bt), f32)
        pm0 = pm + jax.nn.softplus(-llr_i)
        pm1 = pm + jax.nn.softplus(llr_i)
        cand = jnp.concatenate([pm0, pm1], axis=0)
        cbits = jax.lax.bitcast_convert_type(cand, jnp.int32)
        key = (cbits & -16) | cidx
        key8 = _bitonic_16(key, km_masks)[:_L]
        idx8 = key8 & 15
        pidx = idx8 & (_L - 1)
        u_new = (idx8 >= _L).astype(f32)
        pm_new = jnp.zeros((_L, bt), f32)
        for c in range(2 * _L):
            pm_new = jnp.where(idx8 == c, cand[c][None], pm_new)
        pm = pm_new
        masks = [(pidx == src)[:, None, :] for src in range(_L)]
        for s in list(llr_c):
            llr_c[s] = gather_paths(llr_c[s], masks)
        for s in list(ulbuf):
            ulbuf[s] = gather_paths(ulbuf[s], masks)
        recs.append((pidx, u_new))
        return u_new[:, None, :]

    def dec(start, n_sub, s):
        if n_sub == 1:
            return leaf(start)
        half = n_sub // 2
        x = llr_c[s][:, :half]
        y = llr_c[s][:, half:]
        llr_c[s - 1] = _cn_op(x, y)
        u_left = dec(start, half, s - 1)
        ulbuf[s - 1] = u_left
        x = llr_c[s][:, :half]
        y = llr_c[s][:, half:]
        llr_c[s - 1] = _vn_op(x, y, ulbuf[s - 1])
        del llr_c[s]
        u_right = dec(start + half, half, s - 1)
        u_left = ulbuf[s - 1]
        del ulbuf[s - 1]
        p = max(u_left.shape[0], u_right.shape[0])
        u_par = jnp.concatenate(
            [
                jnp.broadcast_to(jnp.mod(u_left + u_right, 2.0), (p, half, bt)),
                jnp.broadcast_to(u_right, (p, half, bt)),
            ],
            axis=1,
        )
        return u_par

    dec(0, _N, _N_STAGES)

    best = jnp.zeros((bt,), jnp.int32)
    bestv = pm[0]
    for l in range(1, _L):
        m = pm[l] < bestv
        bestv = jnp.where(m, pm[l], bestv)
        best = jnp.where(m, l, best)

    bits = [None] * len(recs)
    b = best
    for d in range(len(recs) - 1, -1, -1):
        pidx, u_new = recs[d]
        u_d = jnp.zeros((bt,), f32)
        b_prev = jnp.zeros((bt,), jnp.int32)
        for l in range(_L):
            m = b == l
            u_d = jnp.where(m, u_new[l], u_d)
            b_prev = jnp.where(m, pidx[l], b_prev)
        bits[d] = u_d
        b = b_prev
    out_ref[...] = jnp.stack(bits, axis=0)


@jax.jit
def _decode(llr_t):
    return pl.pallas_call(
        _scl_tile,
        grid=(llr_t.shape[1] // _BATCH_TILE,),
        in_specs=[pl.BlockSpec((_N, _BATCH_TILE), lambda ib: (0, ib))],
        out_specs=pl.BlockSpec((_K_INFO, _BATCH_TILE), lambda ib: (0, ib)),
        out_shape=jax.ShapeDtypeStruct((_K_INFO, llr_t.shape[1]), jnp.float32),
    )(llr_t)


def kernel(llr_ch):
    llr_t = llr_ch.T
    return _decode(llr_t).T


# speedup vs baseline: 118.2687x; 1.3387x over previous
import functools

import jax
import jax.numpy as jnp
import numpy as np
from jax.experimental import pallas as pl

_N = 128
_N_STAGES = 7
_L = 8
_LLR_MAX = 30.0
_BATCH_TILE = 1024

_FROZEN = np.array([1 if bin(i).count("1") <= 3 else 0 for i in range(_N)])
_K_INFO = int(_N - _FROZEN.sum())


def _cn_op(x, y):
    x = jnp.clip(x, -_LLR_MAX, _LLR_MAX)
    y = jnp.clip(y, -_LLR_MAX, _LLR_MAX)
    return jnp.logaddexp(0.0, x + y) - jnp.logaddexp(x, y)


def _vn_op(x, y, u):
    return (1.0 - 2.0 * u) * x + y


def _keepmin_masks(bt):
    pos = jax.lax.broadcasted_iota(jnp.int32, (16, bt), 0)
    masks = {}
    k = 2
    while k <= 16:
        j = k // 2
        while j >= 1:
            masks[(k, j)] = ((pos & j) == 0) == ((pos & k) == 0)
            j //= 2
        k *= 2
    return masks


def _bitonic_16(keys, km_masks):
    n = 16
    k = 2
    while k <= n:
        j = k // 2
        while j >= 1:
            x = keys.reshape(n // (2 * j), 2, j, keys.shape[-1])
            pk = jnp.concatenate([x[:, 1:2], x[:, 0:1]], axis=1).reshape(keys.shape)
            lo = jnp.minimum(keys, pk)
            hi = jnp.maximum(keys, pk)
            keys = jnp.where(km_masks[(k, j)], lo, hi)
            j //= 2
        k *= 2
    return keys


def _scl_tile(llr_ref, out_ref):
    bt = llr_ref.shape[1]
    f32 = jnp.float32

    llr_c = {_N_STAGES: (llr_ref[...][None], None)}
    ulbuf = {}
    lane_l = jax.lax.broadcasted_iota(jnp.int32, (_L, bt), 0)
    pm = jnp.where(lane_l == 0, 0.0, 1.0e9).astype(f32)
    km_masks = _keepmin_masks(bt)
    cidx = jax.lax.broadcasted_iota(jnp.int32, (2 * _L, bt), 0)
    recs = []

    def materialize(buf):
        data, perm = buf
        if perm is None or data.shape[0] == 1:
            return data
        acc = jnp.zeros_like(data)
        for src in range(_L):
            acc = jnp.where((perm == src)[:, None, :], data[src][None], acc)
        return acc

    def compose(perm, pidx, masks2d):
        if perm is None:
            return pidx
        acc = jnp.zeros_like(perm)
        for src in range(_L):
            acc = jnp.where(masks2d[src], perm[src][None], acc)
        return acc

    def leaf(i):
        nonlocal pm
        llr_i = jnp.clip(materialize(llr_c[0])[:, 0, :], -_LLR_MAX, _LLR_MAX)
        del llr_c[0]
        if _FROZEN[i]:
            pm = pm + jax.nn.softplus(-llr_i)
            return jnp.zeros((1, 1, bt), f32)
        pm0 = pm + jax.nn.softplus(-llr_i)
        pm1 = pm + jax.nn.softplus(llr_i)
        cand = jnp.concatenate([pm0, pm1], axis=0)
        cbits = jax.lax.bitcast_convert_type(cand, jnp.int32)
        key = (cbits & -16) | cidx
        key8 = _bitonic_16(key, km_masks)[:_L]
        idx8 = key8 & 15
        pidx = idx8 & (_L - 1)
        u_new = (idx8 >= _L).astype(f32)
        pm_new = jnp.zeros((_L, bt), f32)
        for c in range(2 * _L):
            pm_new = jnp.where(idx8 == c, cand[c][None], pm_new)
        pm = pm_new
        masks2d = [pidx == src for src in range(_L)]
        for s in list(llr_c):
            data, perm = llr_c[s]
            if data.shape[0] > 1:
                llr_c[s] = (data, compose(perm, pidx, masks2d))
        for s in list(ulbuf):
            data, perm = ulbuf[s]
            if data.shape[0] > 1:
                ulbuf[s] = (data, compose(perm, pidx, masks2d))
        recs.append((pidx, u_new))
        return u_new[:, None, :]

    def dec(start, n_sub, s):
        if n_sub == 1:
            return leaf(start)
        half = n_sub // 2
        xy = materialize(llr_c[s])
        x = xy[:, :half]
        y = xy[:, half:]
        llr_c[s - 1] = (_cn_op(x, y), None)
        u_left = dec(start, half, s - 1)
        ulbuf[s - 1] = (u_left, None)
        xy = materialize(llr_c[s])
        x = xy[:, :half]
        y = xy[:, half:]
        llr_c[s - 1] = (_vn_op(x, y, u_left), None)
        del llr_c[s]
        u_right = dec(start + half, half, s - 1)
        u_left = materialize(ulbuf[s - 1])
        del ulbuf[s - 1]
        p = max(u_left.shape[0], u_right.shape[0])
        u_par = jnp.concatenate(
            [
                jnp.broadcast_to(jnp.mod(u_left + u_right, 2.0), (p, half, bt)),
                jnp.broadcast_to(u_right, (p, half, bt)),
            ],
            axis=1,
        )
        return u_par

    dec(0, _N, _N_STAGES)

    best = jnp.zeros((bt,), jnp.int32)
    bestv = pm[0]
    for l in range(1, _L):
        m = pm[l] < bestv
        bestv = jnp.where(m, pm[l], bestv)
        best = jnp.where(m, l, best)

    bits = [None] * len(recs)
    b = best
    for d in range(len(recs) - 1, -1, -1):
        pidx, u_new = recs[d]
        u_d = jnp.zeros((bt,), f32)
        b_prev = jnp.zeros((bt,), jnp.int32)
        for l in range(_L):
            m = b == l
            u_d = jnp.where(m, u_new[l], u_d)
            b_prev = jnp.where(m, pidx[l], b_prev)
        bits[d] = u_d
        b = b_prev
    out_ref[...] = jnp.stack(bits, axis=0)


@jax.jit
def _decode(llr_t):
    return pl.pallas_call(
        _scl_tile,
        grid=(llr_t.shape[1] // _BATCH_TILE,),
        in_specs=[pl.BlockSpec((_N, _BATCH_TILE), lambda ib: (0, ib))],
        out_specs=pl.BlockSpec((_K_INFO, _BATCH_TILE), lambda ib: (0, ib)),
        out_shape=jax.ShapeDtypeStruct((_K_INFO, llr_t.shape[1]), jnp.float32),
    )(llr_t)


def kernel(llr_ch):
    llr_t = llr_ch.T
    return _decode(llr_t).T
